# baseline (speedup 1.0000x reference)
import functools

import jax
import jax.numpy as jnp
from jax import lax
from jax.experimental import pallas as pl
from jax.experimental.pallas import tpu as pltpu
from jax.experimental.pallas import tpu_sc as plsc

F32 = jnp.float32

NUM_CORES = 2
NUM_SUBCORES = 16
LANES = 16
NUM_TILES = NUM_CORES * NUM_SUBCORES

CHUNK = 128


def _tc_prep_body(h_ref, w1_ref, a1_ref, a2_ref, g_ref, s1_ref, s2_ref, m_ref):
    g = jnp.dot(h_ref[...], w1_ref[...], preferred_element_type=F32)
    g_ref[...] = g
    s1 = jnp.dot(g, a1_ref[...], preferred_element_type=F32)
    s2 = jnp.dot(g, a2_ref[...], preferred_element_type=F32)
    s1_ref[...] = s1
    s2_ref[...] = s2
    m = jnp.maximum(jnp.max(s1) + jnp.max(s2), 0.0)
    m_ref[...] = jnp.full((1, 1), m, F32)


def _tc_prep(h2, w1, a1, a2):
    n, f = h2.shape
    return pl.pallas_call(
        _tc_prep_body,
        out_shape=[
            jax.ShapeDtypeStruct((n, f), F32),
            jax.ShapeDtypeStruct((n, 1), F32),
            jax.ShapeDtypeStruct((n, 1), F32),
            jax.ShapeDtypeStruct((1, 1), F32),
        ],
    )(h2, w1, a1, a2)


ABLK = 2000


def _sc_score_body(n_edges, s1_hbm, s2_hbm, m_hbm, src_hbm, dst_hbm, x_hbm,
                   s1_v, s2_v, m_v, sblk, dblk, xblk, sem):
    c = lax.axis_index("c")
    s = lax.axis_index("s")
    tile = c * NUM_SUBCORES + s
    edges_per_tile = n_edges // NUM_TILES

    pltpu.sync_copy(s1_hbm, s1_v)
    pltpu.sync_copy(s2_hbm, s2_v)
    pltpu.sync_copy(m_hbm, m_v)
    mvec = m_v[...]
    base = tile * edges_per_tile

    @pl.loop(0, edges_per_tile // ABLK)
    def _blk(i):
        cb = base + i * ABLK
        pltpu.sync_copy(src_hbm.at[pl.ds(cb, ABLK)], sblk)
        pltpu.sync_copy(dst_hbm.at[pl.ds(cb, ABLK)], dblk)

        for gi in range(ABLK // LANES):
            sv = sblk[pl.ds(gi * LANES, LANES)]
            dv = dblk[pl.ds(gi * LANES, LANES)]
            e = plsc.load_gather(s1_v, [sv]) + plsc.load_gather(s2_v, [dv])
            e = jnp.where(e >= 0.0, e, 0.2 * e)
            xblk[pl.ds(gi * LANES, LANES)] = jnp.exp(e - mvec)

        pltpu.sync_copy(xblk, x_hbm.at[pl.ds(cb, ABLK)])


def _sc_score(s1, s2, m16, src, dst):
    n_nodes = s1.shape[0]
    n_edges = src.shape[0]
    mesh = plsc.VectorSubcoreMesh(core_axis_name="c", subcore_axis_name="s",
                                  num_cores=NUM_CORES,
                                  num_subcores=NUM_SUBCORES)
    body = functools.partial(_sc_score_body, n_edges)
    return pl.kernel(
        body,
        out_type=jax.ShapeDtypeStruct((n_edges,), F32),
        mesh=mesh,
        compiler_params=pltpu.CompilerParams(use_tc_tiling_on_sc=False,
                                             needs_layout_passes=False),
        scratch_types=[
            pltpu.VMEM((n_nodes,), F32),
            pltpu.VMEM((n_nodes,), F32),
            pltpu.VMEM((LANES,), F32),
            pltpu.VMEM((ABLK,), jnp.int32),
            pltpu.VMEM((ABLK,), jnp.int32),
            pltpu.VMEM((ABLK,), F32),
            pltpu.SemaphoreType.DMA,
        ],
    )(s1, s2, m16, src, dst)


def _sc_scatter_body(n_nodes, feat, items_per_tile,
                     g_hbm, src_hbm, dst_hbm, x_hbm, z128_hbm, z16_hbm,
                     out_hbm, dout_hbm,
                     srcv0, srcv1, srcv2, srcv3, dstv0, dstv1, dstv2, dstv3,
                     xv0, xv1, xv2, xv3, rows0, rows1, rx0, rx1,
                     acc, dacc,
                     i0, i1, i2, i3, g0, g1, sc0, sc1):
    c = lax.axis_index("c")
    s = lax.axis_index("s")
    tile = c * NUM_SUBCORES + s
    rows_per_sub = n_nodes // NUM_SUBCORES

    srcv = [srcv0, srcv1, srcv2, srcv3]
    dstv = [dstv0, dstv1, dstv2, dstv3]
    xv = [xv0, xv1, xv2, xv3]
    rows = [rows0, rows1]
    rx = [rx0, rx1]
    isem = [i0, i1, i2, i3]
    gsem = [g0, g1]
    ssem = [sc0, sc1]

    r0 = s * rows_per_sub
    pltpu.sync_copy(z128_hbm.at[pl.ds(r0, rows_per_sub)],
                    acc.at[pl.ds(r0, rows_per_sub)])
    pltpu.sync_copy(z16_hbm.at[pl.ds(r0, rows_per_sub)],
                    dacc.at[pl.ds(r0, rows_per_sub)])
    plsc.subcore_barrier()

    base = tile * items_per_tile * CHUNK

    def issue_idx(j, b):
        cb = base + j * CHUNK
        pltpu.async_copy(src_hbm.at[pl.ds(cb, CHUNK)], srcv[b], isem[b])
        pltpu.async_copy(dst_hbm.at[pl.ds(cb, CHUNK)], dstv[b], isem[b])
        pltpu.async_copy(x_hbm.at[pl.ds(cb, CHUNK)], xv[b], isem[b])

    def wait_idx(b):
        pltpu.make_async_copy(src_hbm.at[pl.ds(0, CHUNK)], srcv[b], isem[b]).wait()
        pltpu.make_async_copy(dst_hbm.at[pl.ds(0, CHUNK)], dstv[b], isem[b]).wait()
        pltpu.make_async_copy(x_hbm.at[pl.ds(0, CHUNK)], xv[b], isem[b]).wait()

    def issue_gather(b4, b2):
        pltpu.async_copy(g_hbm.at[dstv[b4]], rows[b2], gsem[b2])

    def wait_gather(b4, b2):
        pltpu.make_async_copy(g_hbm.at[dstv[b4]], rows[b2], gsem[b2]).wait()

    def issue_scatter(b4, b2):
        pltpu.async_copy(rows[b2], acc.at[srcv[b4]], ssem[b2], add=True)
        pltpu.async_copy(rx[b2], dacc.at[srcv[b4]], ssem[b2], add=True)

    def wait_scatter(b4, b2):
        pltpu.make_async_copy(rows[b2], acc.at[srcv[b4]], ssem[b2]).wait()
        pltpu.make_async_copy(rx[b2], dacc.at[srcv[b4]], ssem[b2]).wait()

    issue_idx(0, 0)
    issue_idx(1, 1)
    wait_idx(0)
    issue_gather(0, 0)

    nsteps = items_per_tile // 4

    @pl.loop(0, nsteps)
    def _step(st):
        j0 = st * 4
        for u in range(4):
            b4 = u
            b2 = u & 1
            j = j0 + u
            wait_gather(b4, b2)
            @pl.when(j < items_per_tile - 1)
            def _():
                wait_idx((u + 1) % 4)

            @pl.when(j > 0)
            def _():
                wait_scatter((u + 3) % 4, 1 - b2)

            @pl.when(j < items_per_tile - 1)
            def _():
                issue_gather((u + 1) % 4, 1 - b2)

            @pl.when(j < items_per_tile - 2)
            def _():
                issue_idx(j + 2, (u + 2) % 4)

            @plsc.parallel_loop(0, CHUNK, unroll=4)
            def _scale(cc):
                xb = plsc.load_gather(xv[b4], [jnp.full((LANES,), cc, jnp.int32)])
                for jj in range(feat // LANES):
                    sl = pl.ds(jj * LANES, LANES)
                    rows[b2][cc, sl] = rows[b2][cc, sl] * xb
                rx[b2][cc, :] = xb

            issue_scatter(b4, b2)

    wait_scatter(3, 1)
    plsc.subcore_barrier()

    pltpu.sync_copy(acc.at[pl.ds(r0, rows_per_sub)],
                    out_hbm.at[c, pl.ds(r0, rows_per_sub)])
    pltpu.sync_copy(dacc.at[pl.ds(r0, rows_per_sub)],
                    dout_hbm.at[c, pl.ds(r0, rows_per_sub)])


def _sc_scatter(g, src_p, dst_p, x_p):
    n_nodes, feat = g.shape
    n_edges_p = src_p.shape[0]
    items_per_tile = n_edges_p // (NUM_TILES * CHUNK)
    mesh = plsc.VectorSubcoreMesh(core_axis_name="c", subcore_axis_name="s",
                                  num_cores=NUM_CORES,
                                  num_subcores=NUM_SUBCORES)
    z128 = jnp.zeros((n_nodes, feat), F32)
    z16 = jnp.zeros((n_nodes, LANES), F32)
    body = functools.partial(_sc_scatter_body, n_nodes, feat, items_per_tile)
    idx_t = [pltpu.VMEM((CHUNK,), jnp.int32)] * 8
    xv_t = [pltpu.VMEM((CHUNK,), F32)] * 4
    return pl.kernel(
        body,
        out_type=[
            jax.ShapeDtypeStruct((NUM_CORES, n_nodes, feat), F32),
            jax.ShapeDtypeStruct((NUM_CORES, n_nodes, LANES), F32),
        ],
        mesh=mesh,
        compiler_params=pltpu.CompilerParams(use_tc_tiling_on_sc=False,
                                             needs_layout_passes=False),
        scratch_types=idx_t + xv_t + [
            pltpu.VMEM((CHUNK, feat), F32),
            pltpu.VMEM((CHUNK, feat), F32),
            pltpu.VMEM((CHUNK, LANES), F32),
            pltpu.VMEM((CHUNK, LANES), F32),
            pltpu.VMEM_SHARED((n_nodes, feat), F32),
            pltpu.VMEM_SHARED((n_nodes, LANES), F32),
        ] + [pltpu.SemaphoreType.DMA] * 8,
    )(g, src_p, dst_p, x_p, z128, z16)


def _tc_final_body(acc_ref, dacc_ref, w2_ref, out_ref):
    hp = acc_ref[0] + acc_ref[1]
    d = dacc_ref[0][:, 0:1] + dacc_ref[1][:, 0:1]
    dinv = jnp.where(d > 0.0, 1.0 / d, 0.0)
    out = jnp.dot(hp * dinv, w2_ref[...], preferred_element_type=F32)
    out_ref[...] = jnp.maximum(out, 0.0)


def _tc_final(acc, dacc, w2):
    n = acc.shape[1]
    out_f = w2.shape[1]
    return pl.pallas_call(
        _tc_final_body,
        out_shape=jax.ShapeDtypeStruct((n, out_f), F32),
    )(acc, dacc, w2)


def kernel(h, edge_index, W_1, W_2, a):
    b, n, f = h.shape
    h2 = h.reshape(n, f)
    a1 = a[:f]
    a2 = a[f:]
    g, s1c, s2c, m = _tc_prep(h2, W_1, a1, a2)
    s1 = s1c.reshape(n)
    s2 = s2c.reshape(n)
    m16 = jnp.broadcast_to(m.reshape(()), (LANES,))
    src = edge_index[0]
    dst = edge_index[1]
    n_edges = src.shape[0]
    x = _sc_score(s1, s2, m16, src, dst)
    quantum = NUM_TILES * CHUNK * 4
    n_pad = (-n_edges) % quantum
    pad_idx = (jnp.arange(n_pad, dtype=jnp.int32)) % n
    src_p = jnp.concatenate([src, pad_idx])
    dst_p = jnp.concatenate([dst, pad_idx])
    x_p = jnp.concatenate([x, jnp.zeros((n_pad,), F32)])
    acc, dacc = _sc_scatter(g, src_p, dst_p, x_p)
    out = _tc_final(acc, dacc, W_2)
    return out.reshape(b, n, W_2.shape[1])


# speedup vs baseline: 1.1321x; 1.1321x over previous
import functools

import jax
import jax.numpy as jnp
from jax import lax
from jax.experimental import pallas as pl
from jax.experimental.pallas import tpu as pltpu
from jax.experimental.pallas import tpu_sc as plsc

F32 = jnp.float32

NUM_CORES = 2
NUM_SUBCORES = 16
LANES = 16
NUM_TILES = NUM_CORES * NUM_SUBCORES

CHUNK = 128


def _tc_prep_body(h_ref, w1_ref, a1_ref, a2_ref, g_ref, s1_ref, s2_ref, m_ref):
    g = jnp.dot(h_ref[...], w1_ref[...], preferred_element_type=F32)
    g_ref[...] = g
    s1 = jnp.dot(g, a1_ref[...], preferred_element_type=F32)
    s2 = jnp.dot(g, a2_ref[...], preferred_element_type=F32)
    s1_ref[...] = s1
    s2_ref[...] = s2
    m = jnp.maximum(jnp.max(s1) + jnp.max(s2), 0.0)
    m_ref[...] = jnp.full((1, 1), m, F32)


def _tc_prep(h2, w1, a1, a2):
    n, f = h2.shape
    return pl.pallas_call(
        _tc_prep_body,
        out_shape=[
            jax.ShapeDtypeStruct((n, f), F32),
            jax.ShapeDtypeStruct((n, 1), F32),
            jax.ShapeDtypeStruct((n, 1), F32),
            jax.ShapeDtypeStruct((1, 1), F32),
        ],
    )(h2, w1, a1, a2)


def _sc_edge_body(n_nodes, feat, items_per_tile, n_edges,
                  g_hbm, src_hbm, dst_hbm, s1_hbm, s2_hbm, m_hbm, iota_hbm,
                  out_hbm, dout_hbm,
                  srcv0, srcv1, srcv2, srcv3, dstv0, dstv1, dstv2, dstv3,
                  sx0, sx1, sx2, sx3, dx0, dx1, dx2, dx3,
                  xv0, xv1, xv2, xv3, rows0, rows1, rx0, rx1,
                  m_v, iota_v, acc, dacc,
                  i0, i1, i2, i3, g0, g1, sc0, sc1):
    c = lax.axis_index("c")
    s = lax.axis_index("s")
    tile = c * NUM_SUBCORES + s
    rows_per_sub = n_nodes // NUM_SUBCORES

    srcv = [srcv0, srcv1, srcv2, srcv3]
    dstv = [dstv0, dstv1, dstv2, dstv3]
    sx = [sx0, sx1, sx2, sx3]
    dx = [dx0, dx1, dx2, dx3]
    xv = [xv0, xv1, xv2, xv3]
    rows = [rows0, rows1]
    rx = [rx0, rx1]
    isem = [i0, i1, i2, i3]
    gsem = [g0, g1]
    ssem = [sc0, sc1]

    pltpu.sync_copy(m_hbm, m_v)
    pltpu.sync_copy(iota_hbm, iota_v)
    mvec = m_v[...]

    zero16 = jnp.zeros((LANES,), F32)

    @plsc.parallel_loop(0, CHUNK, unroll=4)
    def _z(r):
        for jj in range(feat // LANES):
            rows0[r, pl.ds(jj * LANES, LANES)] = zero16
        rx0[r, :] = zero16

    r0 = s * rows_per_sub
    cz = max(d for d in range(1, min(CHUNK, rows_per_sub) + 1)
             if rows_per_sub % d == 0)
    for k in range(rows_per_sub // cz):
        pltpu.sync_copy(rows0.at[pl.ds(0, cz)],
                        acc.at[pl.ds(r0 + k * cz, cz)])
        pltpu.sync_copy(rx0.at[pl.ds(0, cz)],
                        dacc.at[pl.ds(r0 + k * cz, cz)])
    plsc.subcore_barrier()

    base = tile * items_per_tile * CHUNK

    def issue_idx(j, b):
        cb = base + j * CHUNK
        pltpu.async_copy(src_hbm.at[pl.ds(cb, CHUNK)], srcv[b], isem[b])
        pltpu.async_copy(dst_hbm.at[pl.ds(cb, CHUNK)], dstv[b], isem[b])

    def wait_idx(b):
        pltpu.make_async_copy(src_hbm.at[pl.ds(0, CHUNK)], srcv[b], isem[b]).wait()
        pltpu.make_async_copy(dst_hbm.at[pl.ds(0, CHUNK)], dstv[b], isem[b]).wait()

    def issue_sx(b):
        pltpu.async_copy(s1_hbm.at[srcv[b]], sx[b], isem[b])
        pltpu.async_copy(s2_hbm.at[dstv[b]], dx[b], isem[b])

    def wait_sx(b):
        pltpu.make_async_copy(s1_hbm.at[srcv[b]], sx[b], isem[b]).wait()
        pltpu.make_async_copy(s2_hbm.at[dstv[b]], dx[b], isem[b]).wait()

    def compute_x(b, cb):
        for gi in range(CHUNK // LANES):
            sl = pl.ds(gi * LANES, LANES)
            e = sx[b][sl] + dx[b][sl]
            e = jnp.where(e >= 0.0, e, 0.2 * e)
            x = jnp.exp(e - mvec)
            gidx = iota_v[sl] + cb
            xv[b][sl] = jnp.where(gidx < n_edges, x, 0.0)

    def issue_gather(b4, b2):
        pltpu.async_copy(g_hbm.at[dstv[b4]], rows[b2], gsem[b2])

    def wait_gather(b4, b2):
        pltpu.make_async_copy(g_hbm.at[dstv[b4]], rows[b2], gsem[b2]).wait()

    def issue_scatter(b4, b2):
        pltpu.async_copy(rows[b2], acc.at[srcv[b4]], ssem[b2], add=True)
        pltpu.async_copy(rx[b2], dacc.at[srcv[b4]], ssem[b2], add=True)

    def wait_scatter(b4, b2):
        pltpu.make_async_copy(rows[b2], acc.at[srcv[b4]], ssem[b2]).wait()
        pltpu.make_async_copy(rx[b2], dacc.at[srcv[b4]], ssem[b2]).wait()

    issue_idx(0, 0)
    issue_idx(1, 1)
    wait_idx(0)
    issue_sx(0)
    issue_gather(0, 0)

    nsteps = items_per_tile // 4

    @pl.loop(0, nsteps)
    def _step(st):
        j0 = st * 4
        for u in range(4):
            b4 = u
            b2 = u & 1
            j = j0 + u
            wait_gather(b4, b2)
            @pl.when(j < items_per_tile - 1)
            def _():
                wait_idx((u + 1) % 4)
                issue_sx((u + 1) % 4)

            @pl.when(j > 0)
            def _():
                wait_scatter((u + 3) % 4, 1 - b2)

            @pl.when(j < items_per_tile - 1)
            def _():
                issue_gather((u + 1) % 4, 1 - b2)

            @pl.when(j < items_per_tile - 2)
            def _():
                issue_idx(j + 2, (u + 2) % 4)

            wait_sx(b4)
            compute_x(b4, base + j * CHUNK)

            @plsc.parallel_loop(0, CHUNK, unroll=4)
            def _scale(cc):
                xb = plsc.load_gather(xv[b4], [jnp.full((LANES,), cc, jnp.int32)])
                for jj in range(feat // LANES):
                    sl = pl.ds(jj * LANES, LANES)
                    rows[b2][cc, sl] = rows[b2][cc, sl] * xb
                rx[b2][cc, :] = xb

            issue_scatter(b4, b2)

    wait_scatter(3, 1)
    plsc.subcore_barrier()

    pltpu.sync_copy(acc.at[pl.ds(r0, rows_per_sub)],
                    out_hbm.at[c, pl.ds(r0, rows_per_sub)])
    pltpu.sync_copy(dacc.at[pl.ds(r0, rows_per_sub)],
                    dout_hbm.at[c, pl.ds(r0, rows_per_sub)])


def _sc_edge(g, src_p, dst_p, s1, s2, m16, iota, n_edges):
    n_nodes, feat = g.shape
    n_edges_p = src_p.shape[0]
    items_per_tile = n_edges_p // (NUM_TILES * CHUNK)
    mesh = plsc.VectorSubcoreMesh(core_axis_name="c", subcore_axis_name="s",
                                  num_cores=NUM_CORES,
                                  num_subcores=NUM_SUBCORES)
    body = functools.partial(_sc_edge_body, n_nodes, feat, items_per_tile,
                             n_edges)
    idx_t = [pltpu.VMEM((CHUNK,), jnp.int32)] * 8
    sx_t = [pltpu.VMEM((CHUNK,), F32)] * 8
    xv_t = [pltpu.VMEM((CHUNK,), F32)] * 4
    return pl.kernel(
        body,
        out_type=[
            jax.ShapeDtypeStruct((NUM_CORES, n_nodes, feat), F32),
            jax.ShapeDtypeStruct((NUM_CORES, n_nodes, LANES), F32),
        ],
        mesh=mesh,
        compiler_params=pltpu.CompilerParams(use_tc_tiling_on_sc=False,
                                             needs_layout_passes=False),
        scratch_types=idx_t + sx_t + xv_t + [
            pltpu.VMEM((CHUNK, feat), F32),
            pltpu.VMEM((CHUNK, feat), F32),
            pltpu.VMEM((CHUNK, LANES), F32),
            pltpu.VMEM((CHUNK, LANES), F32),
            pltpu.VMEM((LANES,), F32),
            pltpu.VMEM((CHUNK,), jnp.int32),
            pltpu.VMEM_SHARED((n_nodes, feat), F32),
            pltpu.VMEM_SHARED((n_nodes, LANES), F32),
        ] + [pltpu.SemaphoreType.DMA] * 8,
    )(g, src_p, dst_p, s1, s2, m16, iota)


def _tc_final_body(acc_ref, dacc_ref, w2_ref, out_ref):
    hp = acc_ref[0] + acc_ref[1]
    d = dacc_ref[0][:, 0:1] + dacc_ref[1][:, 0:1]
    dinv = jnp.where(d > 0.0, 1.0 / d, 0.0)
    out = jnp.dot(hp * dinv, w2_ref[...], preferred_element_type=F32)
    out_ref[...] = jnp.maximum(out, 0.0)


def _tc_final(acc, dacc, w2):
    n = acc.shape[1]
    out_f = w2.shape[1]
    return pl.pallas_call(
        _tc_final_body,
        out_shape=jax.ShapeDtypeStruct((n, out_f), F32),
    )(acc, dacc, w2)


def kernel(h, edge_index, W_1, W_2, a):
    b, n, f = h.shape
    h2 = h.reshape(n, f)
    a1 = a[:f]
    a2 = a[f:]
    g, s1c, s2c, m = _tc_prep(h2, W_1, a1, a2)
    s1 = s1c.reshape(n)
    s2 = s2c.reshape(n)
    m16 = jnp.broadcast_to(m.reshape(()), (LANES,))
    src = edge_index[0]
    dst = edge_index[1]
    n_edges = src.shape[0]
    quantum = NUM_TILES * CHUNK * 4
    n_pad = (-n_edges) % quantum
    pad_idx = (jnp.arange(n_pad, dtype=jnp.int32)) % n
    src_p = jnp.concatenate([src, pad_idx])
    dst_p = jnp.concatenate([dst, pad_idx])
    iota = jnp.arange(CHUNK, dtype=jnp.int32)
    acc, dacc = _sc_edge(g, src_p, dst_p, s1, s2, m16, iota, n_edges)
    out = _tc_final(acc, dacc, W_2)
    return out.reshape(b, n, W_2.shape[1])
